# baseline (speedup 1.0000x reference)
import functools

import jax
import jax.numpy as jnp
from jax import lax
from jax.experimental import pallas as pl
from jax.experimental.pallas import tpu as pltpu
from jax.experimental.pallas import tpu_sc as plsc

N = 10000
N_PAD = 10240
PAD_IDX = N
E = 320000
E_PAD = 327680
E2 = 100000
E2_PAD = 102400
EBLK = 128
NC, NS = 2, 16
RPT = N_PAD // NS
CHUNK = 16

_MESH = plsc.VectorSubcoreMesh(
    core_axis_name="c", subcore_axis_name="s", num_cores=NC, num_subcores=NS
)

F32 = jnp.float32


NBLK = E_PAD // (NC * NS) // EBLK
NCHUNK = NBLK // CHUNK
NBLK2 = E2_PAD // NS // EBLK


def _deg_body(dst_hbm, deg_out, deg_sh, ones_v, idx_v, zrow_v):
    c = lax.axis_index("c")
    s = lax.axis_index("s")
    for j in range(EBLK // 16):
        ones_v[pl.ds(j * 16, 16)] = jnp.ones((16,), F32)

    def zfill(i, carry):
        zrow_v[pl.ds(i * 16, 16)] = jnp.zeros((16,), F32)
        return carry

    lax.fori_loop(0, RPT // 16, zfill, 0)
    pltpu.sync_copy(zrow_v, deg_sh.at[pl.ds(s * RPT, RPT)])
    pltpu.sync_copy(dst_hbm.at[c * NS + s], idx_v)
    plsc.subcore_barrier()

    def body(i, carry):
        pltpu.sync_copy(ones_v, deg_sh.at[idx_v.at[i]], add=True)
        return carry

    lax.fori_loop(0, NBLK, body, 0)
    plsc.subcore_barrier()
    pltpu.sync_copy(deg_sh.at[pl.ds(s * RPT, RPT)],
                    deg_out.at[c, pl.ds(s * RPT, RPT)])


_deg = pl.kernel(
    _deg_body,
    out_type=jax.ShapeDtypeStruct((NC, N_PAD), F32),
    mesh=_MESH,
    scratch_types=[
        pltpu.VMEM_SHARED((N_PAD,), F32),
        pltpu.VMEM((EBLK,), F32),
        pltpu.VMEM((NBLK, EBLK), jnp.int32),
        pltpu.VMEM((RPT,), F32),
    ],
)


def _agg_body(src_hbm, dst_hbm, g_hbm, out_hbm, acc_sh,
              r0, r1, cs0, cs1, cd0, cd1,
              gsem0, gsem1, ssem0, ssem1, dsem0, dsem1):
    c = lax.axis_index("c")
    s = lax.axis_index("s")
    rows = (r0, r1)
    gsems = (gsem0, gsem1)
    csrc = (cs0, cs1)
    cdst = (cd0, cd1)
    ssems = (ssem0, ssem1)
    dsems = (dsem0, dsem1)
    wid = c * NS + s

    def zfill(i, carry):
        for j in range(128 // 16):
            r0[i, pl.ds(j * 16, 16)] = jnp.zeros((16,), F32)
        return carry

    lax.fori_loop(0, EBLK, zfill, 0)
    for k in range(RPT // EBLK):
        pltpu.sync_copy(r0, acc_sh.at[pl.ds(s * RPT + k * EBLK, EBLK)])
    pltpu.sync_copy(src_hbm.at[wid, pl.ds(0, CHUNK)], cs0)
    pltpu.sync_copy(dst_hbm.at[wid, pl.ds(0, CHUNK)], cd0)
    plsc.subcore_barrier()
    pltpu.async_copy(g_hbm.at[cs0.at[0]], r0, gsem0)

    def chunk_body(k, carry):
        for kp in range(2):
            @pl.when(k % 2 == kp)
            def _():
                @pl.when(k + 1 < NCHUNK)
                def _():
                    off = pl.multiple_of((k + 1) * CHUNK, CHUNK)
                    pltpu.async_copy(src_hbm.at[wid, pl.ds(off, CHUNK)],
                                     csrc[1 - kp], ssems[1 - kp])
                    pltpu.async_copy(dst_hbm.at[wid, pl.ds(off, CHUNK)],
                                     cdst[1 - kp], dsems[1 - kp])
                for j in range(CHUNK):
                    p = j % 2
                    if j + 1 < CHUNK:
                        pltpu.async_copy(g_hbm.at[csrc[kp].at[j + 1]],
                                         rows[1 - p], gsems[1 - p])
                    else:
                        @pl.when(k + 1 < NCHUNK)
                        def _():
                            pltpu.make_async_copy(
                                src_hbm.at[wid, pl.ds(0, CHUNK)],
                                csrc[1 - kp], ssems[1 - kp]).wait()
                            pltpu.make_async_copy(
                                dst_hbm.at[wid, pl.ds(0, CHUNK)],
                                cdst[1 - kp], dsems[1 - kp]).wait()
                            pltpu.async_copy(g_hbm.at[csrc[1 - kp].at[0]],
                                             rows[1 - p], gsems[1 - p])
                    pltpu.make_async_copy(g_hbm.at[csrc[kp].at[j]],
                                          rows[p], gsems[p]).wait()
                    pltpu.sync_copy(rows[p], acc_sh.at[cdst[kp].at[j]],
                                    add=True)
        return carry

    lax.fori_loop(0, NCHUNK, chunk_body, 0)
    plsc.subcore_barrier()
    pltpu.sync_copy(acc_sh.at[pl.ds(s * RPT, RPT)],
                    out_hbm.at[c, pl.ds(s * RPT, RPT)])


_agg = pl.kernel(
    _agg_body,
    out_type=jax.ShapeDtypeStruct((NC, N_PAD, 128), F32),
    mesh=_MESH,
    scratch_types=[
        pltpu.VMEM_SHARED((N_PAD, 128), F32),
        pltpu.VMEM((EBLK, 128), F32),
        pltpu.VMEM((EBLK, 128), F32),
        pltpu.VMEM((CHUNK, EBLK), jnp.int32),
        pltpu.VMEM((CHUNK, EBLK), jnp.int32),
        pltpu.VMEM((CHUNK, EBLK), jnp.int32),
        pltpu.VMEM((CHUNK, EBLK), jnp.int32),
        pltpu.SemaphoreType.DMA,
        pltpu.SemaphoreType.DMA,
        pltpu.SemaphoreType.DMA,
        pltpu.SemaphoreType.DMA,
        pltpu.SemaphoreType.DMA,
        pltpu.SemaphoreType.DMA,
    ],
)


NBLK2C = E2_PAD // (NC * NS) // EBLK


def _dec_body(e0_hbm, e1_hbm, t_hbm, bm1_hbm, wm2_hbm, out_hbm,
              ts0, ts1, td0, td1, o0, o1, i0_v, i1_v, bm1_v, wm2_v,
              sa0, sa1, sb0, sb1, w0, w1):
    c = lax.axis_index("c")
    s = lax.axis_index("s")
    ts = (ts0, ts1)
    td = (td0, td1)
    outs = (o0, o1)
    sas = (sa0, sa1)
    sbs = (sb0, sb1)
    ws = (w0, w1)
    base = (c * NS + s) * (NBLK2C * EBLK)

    pltpu.sync_copy(bm1_hbm, bm1_v)
    pltpu.sync_copy(wm2_hbm, wm2_v)
    pltpu.sync_copy(e0_hbm.at[c, s], i0_v)
    pltpu.sync_copy(e1_hbm.at[c, s], i1_v)
    pltpu.async_copy(t_hbm.at[i0_v.at[0]], ts0, sa0)
    pltpu.async_copy(t_hbm.at[i1_v.at[0]], td0, sb0)

    def body(i, carry):
        for p in range(2):
            @pl.when(i % 2 == p)
            def _():
                @pl.when(i + 1 < NBLK2C)
                def _():
                    pltpu.async_copy(t_hbm.at[i0_v.at[i + 1]],
                                     ts[1 - p], sas[1 - p])
                    pltpu.async_copy(t_hbm.at[i1_v.at[i + 1]],
                                     td[1 - p], sbs[1 - p])
                pltpu.make_async_copy(t_hbm.at[i0_v.at[i]],
                                      ts[p], sas[p]).wait()
                pltpu.make_async_copy(t_hbm.at[i1_v.at[i]],
                                      td[p], sbs[p]).wait()

                @pl.when(i >= 2)
                def _():
                    pltpu.make_async_copy(
                        outs[p],
                        out_hbm.at[pl.ds(base, EBLK)], ws[p]).wait()

                def pair(r, carry2):
                    acc = jnp.zeros((16,), F32)
                    for k in range(4):
                        a = ts[p][r, pl.ds(k * 16, 16)]
                        b = td[p][r, pl.ds(64 + k * 16, 16)]
                        h = jnp.maximum(a + b + bm1_v[pl.ds(k * 16, 16)], 0.0)
                        acc = acc + h * wm2_v[pl.ds(k * 16, 16)]
                    outs[p][r, pl.ds(0, 16)] = acc
                    return carry2

                lax.fori_loop(0, EBLK, pair, 0)
                pltpu.async_copy(
                    outs[p], out_hbm.at[pl.ds(base + i * EBLK, EBLK)], ws[p])
        return carry

    lax.fori_loop(0, NBLK2C, body, 0)
    pltpu.make_async_copy(o0, out_hbm.at[pl.ds(base, EBLK)], w0).wait()
    pltpu.make_async_copy(o1, out_hbm.at[pl.ds(base, EBLK)], w1).wait()


_dec = pl.kernel(
    _dec_body,
    out_type=jax.ShapeDtypeStruct((E2_PAD, 16), F32),
    mesh=_MESH,
    scratch_types=[
        pltpu.VMEM((EBLK, 128), F32),
        pltpu.VMEM((EBLK, 128), F32),
        pltpu.VMEM((EBLK, 128), F32),
        pltpu.VMEM((EBLK, 128), F32),
        pltpu.VMEM((EBLK, 16), F32),
        pltpu.VMEM((EBLK, 16), F32),
        pltpu.VMEM((NBLK2C, EBLK), jnp.int32),
        pltpu.VMEM((NBLK2C, EBLK), jnp.int32),
        pltpu.VMEM((64,), F32),
        pltpu.VMEM((64,), F32),
        pltpu.SemaphoreType.DMA,
        pltpu.SemaphoreType.DMA,
        pltpu.SemaphoreType.DMA,
        pltpu.SemaphoreType.DMA,
        pltpu.SemaphoreType.DMA,
        pltpu.SemaphoreType.DMA,
    ],
)


_BT = 1024


def _tc1_body(x_ref, w_ref, dinv_ref, o_ref):
    h = jnp.dot(x_ref[...], w_ref[...], preferred_element_type=F32)
    o_ref[...] = h * dinv_ref[...]


def _tc_layer_body(o_ref, g_ref, dinv_ref, b_ref, w_ref, out_ref):
    z = dinv_ref[...] * (o_ref[0] + o_ref[1] + g_ref[...]) + b_ref[...]
    z = jnp.maximum(z, 0.0)
    out_ref[...] = dinv_ref[...] * jnp.dot(
        z, w_ref[...], preferred_element_type=F32)


def _tc4_body(o_ref, g_ref, dinv_ref, b_ref, wm_ref, t_ref):
    z = dinv_ref[...] * (o_ref[0] + o_ref[1] + g_ref[...]) + b_ref[...]
    t_ref[...] = jnp.dot(z, wm_ref[...], preferred_element_type=F32)


def _tc5_body(s_ref, bm2_ref, o_ref):
    o_ref[...] = jnp.sum(s_ref[...], axis=1, keepdims=True) + bm2_ref[...]


def _tc1(x_p, W1, dinv):
    return pl.pallas_call(
        _tc1_body,
        grid=(N_PAD // _BT,),
        in_specs=[
            pl.BlockSpec((_BT, 128), lambda i: (i, 0)),
            pl.BlockSpec((128, 128), lambda i: (0, 0)),
            pl.BlockSpec((_BT, 1), lambda i: (i, 0)),
        ],
        out_specs=pl.BlockSpec((_BT, 128), lambda i: (i, 0)),
        out_shape=jax.ShapeDtypeStruct((N_PAD, 128), F32),
    )(x_p, W1, dinv)


def _tc_layer(o, g, dinv, b, W):
    return pl.pallas_call(
        _tc_layer_body,
        grid=(N_PAD // _BT,),
        in_specs=[
            pl.BlockSpec((NC, _BT, 128), lambda i: (0, i, 0)),
            pl.BlockSpec((_BT, 128), lambda i: (i, 0)),
            pl.BlockSpec((_BT, 1), lambda i: (i, 0)),
            pl.BlockSpec((1, 128), lambda i: (0, 0)),
            pl.BlockSpec((128, 128), lambda i: (0, 0)),
        ],
        out_specs=pl.BlockSpec((_BT, 128), lambda i: (i, 0)),
        out_shape=jax.ShapeDtypeStruct((N_PAD, 128), F32),
    )(o, g, dinv, b, W)


def _tc4(o, g, dinv, b, Wm):
    return pl.pallas_call(
        _tc4_body,
        grid=(N_PAD // _BT,),
        in_specs=[
            pl.BlockSpec((NC, _BT, 128), lambda i: (0, i, 0)),
            pl.BlockSpec((_BT, 128), lambda i: (i, 0)),
            pl.BlockSpec((_BT, 1), lambda i: (i, 0)),
            pl.BlockSpec((1, 128), lambda i: (0, 0)),
            pl.BlockSpec((128, 128), lambda i: (0, 0)),
        ],
        out_specs=pl.BlockSpec((_BT, 128), lambda i: (i, 0)),
        out_shape=jax.ShapeDtypeStruct((N_PAD, 128), F32),
    )(o, g, dinv, b, Wm)


def _tc5(S16, bm2):
    B2 = 4096
    return pl.pallas_call(
        _tc5_body,
        grid=(E2_PAD // B2,),
        in_specs=[
            pl.BlockSpec((B2, 16), lambda i: (i, 0)),
            pl.BlockSpec((1, 1), lambda i: (0, 0)),
        ],
        out_specs=pl.BlockSpec((B2, 1), lambda i: (i, 0)),
        out_shape=jax.ShapeDtypeStruct((E2_PAD, 1), F32),
    )(S16, bm2)


def kernel(x, edge_index, edge_label_index,
           W1, b1, W2, b2, W3, b3, Wm1, bm1, Wm2, bm2):
    ei = edge_index.astype(jnp.int32)
    eli = edge_label_index.astype(jnp.int32)

    pad_e = PAD_IDX + (jnp.arange(E_PAD - E, dtype=jnp.int32) % (N_PAD - N))
    src = jnp.concatenate([ei[0], pad_e]).reshape(NC * NS, NBLK, EBLK)
    dst = jnp.concatenate([ei[1], pad_e]).reshape(NC * NS, NBLK, EBLK)
    eli_p = jnp.concatenate(
        [eli, jnp.full((2, E2_PAD - E2), PAD_IDX, jnp.int32)],
        axis=1).reshape(2, NC, NS, NBLK2C, EBLK)
    x_p = jnp.pad(x, ((0, N_PAD - N), (0, 0)))
    W3p = jnp.pad(W3, ((0, 0), (0, 64)))
    b3p = jnp.pad(b3, (0, 64)).reshape(1, 128)
    Wmp = jnp.pad(jnp.concatenate([Wm1[:64], Wm1[64:]], axis=1),
                  ((0, 64), (0, 0)))

    degs = _deg(dst)
    dinv = (1.0 / jnp.sqrt(1.0 + degs[0] + degs[1])).reshape(N_PAD, 1)

    g1 = _tc1(x_p, W1, dinv)
    o1 = _agg(src, dst, g1)
    g2 = _tc_layer(o1, g1, dinv, b1.reshape(1, 128), W2)
    o2 = _agg(src, dst, g2)
    g3 = _tc_layer(o2, g2, dinv, b2.reshape(1, 128), W3p)
    o3 = _agg(src, dst, g3)
    T = _tc4(o3, g3, dinv, b3p, Wmp)
    S16 = _dec(eli_p[0], eli_p[1], T, bm1, Wm2[:, 0])
    logits = _tc5(S16, bm2.reshape(1, 1))
    return logits[:E2, 0]


# speedup vs baseline: 1.4901x; 1.4901x over previous
import functools

import jax
import jax.numpy as jnp
from jax import lax
from jax.experimental import pallas as pl
from jax.experimental.pallas import tpu as pltpu
from jax.experimental.pallas import tpu_sc as plsc

N = 10000
N_PAD = 10240
PAD_IDX = N
E = 320000
E_PAD = 327680
E2 = 100000
E2_PAD = 102400
EBLK = 128
NC, NS = 2, 16
RPT = N_PAD // NS
CHUNK = 16

_MESH = plsc.VectorSubcoreMesh(
    core_axis_name="c", subcore_axis_name="s", num_cores=NC, num_subcores=NS
)

F32 = jnp.float32


NBLK = E_PAD // (NC * NS) // EBLK
NCHUNK = NBLK // CHUNK
NBLK2 = E2_PAD // NS // EBLK


def _deg_body(dst_hbm, deg_out, deg_sh, ones_v, idx_v, zrow_v):
    c = lax.axis_index("c")
    s = lax.axis_index("s")
    for j in range(EBLK // 16):
        ones_v[pl.ds(j * 16, 16)] = jnp.ones((16,), F32)

    def zfill(i, carry):
        zrow_v[pl.ds(i * 16, 16)] = jnp.zeros((16,), F32)
        return carry

    lax.fori_loop(0, RPT // 16, zfill, 0)
    pltpu.sync_copy(zrow_v, deg_sh.at[pl.ds(s * RPT, RPT)])
    pltpu.sync_copy(dst_hbm.at[c * NS + s], idx_v)
    plsc.subcore_barrier()

    def body(i, carry):
        pltpu.sync_copy(ones_v, deg_sh.at[idx_v.at[i]], add=True)
        return carry

    lax.fori_loop(0, NBLK, body, 0)
    plsc.subcore_barrier()
    pltpu.sync_copy(deg_sh.at[pl.ds(s * RPT, RPT)],
                    deg_out.at[c, pl.ds(s * RPT, RPT)])


_deg = pl.kernel(
    _deg_body,
    out_type=jax.ShapeDtypeStruct((NC, N_PAD), F32),
    mesh=_MESH,
    scratch_types=[
        pltpu.VMEM_SHARED((N_PAD,), F32),
        pltpu.VMEM((EBLK,), F32),
        pltpu.VMEM((NBLK, EBLK), jnp.int32),
        pltpu.VMEM((RPT,), F32),
    ],
)


def _agg_body(src_hbm, dst_hbm, g_hbm, out_hbm, acc_sh,
              r0, r1, cs0, cs1, cd0, cd1,
              gsem0, gsem1, ssem0, ssem1, dsem0, dsem1):
    c = lax.axis_index("c")
    s = lax.axis_index("s")
    rows = (r0, r1)
    gsems = (gsem0, gsem1)
    csrc = (cs0, cs1)
    cdst = (cd0, cd1)
    ssems = (ssem0, ssem1)
    dsems = (dsem0, dsem1)
    wid = c * NS + s

    def zfill(i, carry):
        for j in range(128 // 16):
            r0[i, pl.ds(j * 16, 16)] = jnp.zeros((16,), F32)
        return carry

    lax.fori_loop(0, EBLK, zfill, 0)
    for k in range(RPT // EBLK):
        pltpu.sync_copy(r0, acc_sh.at[pl.ds(s * RPT + k * EBLK, EBLK)])
    pltpu.sync_copy(src_hbm.at[wid, pl.ds(0, CHUNK)], cs0)
    pltpu.sync_copy(dst_hbm.at[wid, pl.ds(0, CHUNK)], cd0)
    plsc.subcore_barrier()
    pltpu.async_copy(g_hbm.at[cs0.at[0]], r0, gsem0)

    def chunk_body(k, carry):
        for kp in range(2):
            @pl.when(k % 2 == kp)
            def _():
                @pl.when(k + 1 < NCHUNK)
                def _():
                    off = pl.multiple_of((k + 1) * CHUNK, CHUNK)
                    pltpu.async_copy(src_hbm.at[wid, pl.ds(off, CHUNK)],
                                     csrc[1 - kp], ssems[1 - kp])
                    pltpu.async_copy(dst_hbm.at[wid, pl.ds(off, CHUNK)],
                                     cdst[1 - kp], dsems[1 - kp])
                for j in range(CHUNK):
                    p = j % 2
                    if j + 1 < CHUNK:
                        pltpu.async_copy(g_hbm.at[csrc[kp].at[j + 1]],
                                         rows[1 - p], gsems[1 - p])
                    else:
                        @pl.when(k + 1 < NCHUNK)
                        def _():
                            pltpu.make_async_copy(
                                src_hbm.at[wid, pl.ds(0, CHUNK)],
                                csrc[1 - kp], ssems[1 - kp]).wait()
                            pltpu.make_async_copy(
                                dst_hbm.at[wid, pl.ds(0, CHUNK)],
                                cdst[1 - kp], dsems[1 - kp]).wait()
                            pltpu.async_copy(g_hbm.at[csrc[1 - kp].at[0]],
                                             rows[1 - p], gsems[1 - p])
                    pltpu.make_async_copy(g_hbm.at[csrc[kp].at[j]],
                                          rows[p], gsems[p]).wait()
                    pltpu.sync_copy(rows[p], acc_sh.at[cdst[kp].at[j]],
                                    add=True)
        return carry

    lax.fori_loop(0, NCHUNK, chunk_body, 0)
    plsc.subcore_barrier()
    pltpu.sync_copy(acc_sh.at[pl.ds(s * RPT, RPT)],
                    out_hbm.at[c, pl.ds(s * RPT, RPT)])


_agg = pl.kernel(
    _agg_body,
    out_type=jax.ShapeDtypeStruct((NC, N_PAD, 128), F32),
    mesh=_MESH,
    scratch_types=[
        pltpu.VMEM_SHARED((N_PAD, 128), F32),
        pltpu.VMEM((EBLK, 128), F32),
        pltpu.VMEM((EBLK, 128), F32),
        pltpu.VMEM((CHUNK, EBLK), jnp.int32),
        pltpu.VMEM((CHUNK, EBLK), jnp.int32),
        pltpu.VMEM((CHUNK, EBLK), jnp.int32),
        pltpu.VMEM((CHUNK, EBLK), jnp.int32),
        pltpu.SemaphoreType.DMA,
        pltpu.SemaphoreType.DMA,
        pltpu.SemaphoreType.DMA,
        pltpu.SemaphoreType.DMA,
        pltpu.SemaphoreType.DMA,
        pltpu.SemaphoreType.DMA,
    ],
)


NBLK2C = E2_PAD // (NC * NS) // EBLK


def _dec_body(e0_hbm, e1_hbm, t_hbm, bm1_hbm, wm2_hbm, out_hbm,
              ts0, ts1, td0, td1, o0, o1, i0_v, i1_v, bm1_v, wm2_v,
              sa0, sa1, sb0, sb1, w0, w1):
    c = lax.axis_index("c")
    s = lax.axis_index("s")
    ts = (ts0, ts1)
    td = (td0, td1)
    outs = (o0, o1)
    sas = (sa0, sa1)
    sbs = (sb0, sb1)
    ws = (w0, w1)
    base = (c * NS + s) * (NBLK2C * EBLK)

    pltpu.sync_copy(bm1_hbm, bm1_v)
    pltpu.sync_copy(wm2_hbm, wm2_v)
    pltpu.sync_copy(e0_hbm.at[c, s], i0_v)
    pltpu.sync_copy(e1_hbm.at[c, s], i1_v)
    pltpu.async_copy(t_hbm.at[i0_v.at[0]], ts0, sa0)
    pltpu.async_copy(t_hbm.at[i1_v.at[0]], td0, sb0)

    def body(i, carry):
        for p in range(2):
            @pl.when(i % 2 == p)
            def _():
                @pl.when(i + 1 < NBLK2C)
                def _():
                    pltpu.async_copy(t_hbm.at[i0_v.at[i + 1]],
                                     ts[1 - p], sas[1 - p])
                    pltpu.async_copy(t_hbm.at[i1_v.at[i + 1]],
                                     td[1 - p], sbs[1 - p])
                pltpu.make_async_copy(t_hbm.at[i0_v.at[i]],
                                      ts[p], sas[p]).wait()
                pltpu.make_async_copy(t_hbm.at[i1_v.at[i]],
                                      td[p], sbs[p]).wait()

                @pl.when(i >= 2)
                def _():
                    pltpu.make_async_copy(
                        outs[p],
                        out_hbm.at[pl.ds(base, EBLK)], ws[p]).wait()

                def pair(r, carry2):
                    acc = jnp.zeros((16,), F32)
                    for k in range(4):
                        a = ts[p][r, pl.ds(k * 16, 16)]
                        b = td[p][r, pl.ds(64 + k * 16, 16)]
                        h = jnp.maximum(a + b + bm1_v[pl.ds(k * 16, 16)], 0.0)
                        acc = acc + h * wm2_v[pl.ds(k * 16, 16)]
                    outs[p][r, pl.ds(0, 16)] = acc
                    return carry2

                lax.fori_loop(0, EBLK, pair, 0)
                pltpu.async_copy(
                    outs[p], out_hbm.at[pl.ds(base + i * EBLK, EBLK)], ws[p])
        return carry

    lax.fori_loop(0, NBLK2C, body, 0)
    pltpu.make_async_copy(o0, out_hbm.at[pl.ds(base, EBLK)], w0).wait()
    pltpu.make_async_copy(o1, out_hbm.at[pl.ds(base, EBLK)], w1).wait()


_dec = pl.kernel(
    _dec_body,
    out_type=jax.ShapeDtypeStruct((E2_PAD, 16), F32),
    mesh=_MESH,
    scratch_types=[
        pltpu.VMEM((EBLK, 128), F32),
        pltpu.VMEM((EBLK, 128), F32),
        pltpu.VMEM((EBLK, 128), F32),
        pltpu.VMEM((EBLK, 128), F32),
        pltpu.VMEM((EBLK, 16), F32),
        pltpu.VMEM((EBLK, 16), F32),
        pltpu.VMEM((NBLK2C, EBLK), jnp.int32),
        pltpu.VMEM((NBLK2C, EBLK), jnp.int32),
        pltpu.VMEM((64,), F32),
        pltpu.VMEM((64,), F32),
        pltpu.SemaphoreType.DMA,
        pltpu.SemaphoreType.DMA,
        pltpu.SemaphoreType.DMA,
        pltpu.SemaphoreType.DMA,
        pltpu.SemaphoreType.DMA,
        pltpu.SemaphoreType.DMA,
    ],
)


_BT = 1024


def _tc1_body(x_ref, w_ref, dinv_ref, o_ref):
    h = jnp.dot(x_ref[...], w_ref[...], preferred_element_type=F32)
    o_ref[...] = h * dinv_ref[...]


def _tc_layer_body(o_ref, g_ref, dinv_ref, b_ref, w_ref, out_ref):
    z = dinv_ref[...] * (o_ref[0] + o_ref[1] + g_ref[...]) + b_ref[...]
    z = jnp.maximum(z, 0.0)
    out_ref[...] = dinv_ref[...] * jnp.dot(
        z, w_ref[...], preferred_element_type=F32)


def _tc4_body(o_ref, g_ref, dinv_ref, b_ref, wm_ref, t_ref):
    z = dinv_ref[...] * (o_ref[0] + o_ref[1] + g_ref[...]) + b_ref[...]
    t_ref[...] = jnp.dot(z, wm_ref[...], preferred_element_type=F32)


def _tc5_body(s_ref, bm2_ref, o_ref):
    o_ref[...] = jnp.sum(s_ref[...], axis=1, keepdims=True) + bm2_ref[...]


def _tc1(x_p, W1, dinv):
    return pl.pallas_call(
        _tc1_body,
        grid=(N_PAD // _BT,),
        in_specs=[
            pl.BlockSpec((_BT, 128), lambda i: (i, 0)),
            pl.BlockSpec((128, 128), lambda i: (0, 0)),
            pl.BlockSpec((_BT, 1), lambda i: (i, 0)),
        ],
        out_specs=pl.BlockSpec((_BT, 128), lambda i: (i, 0)),
        out_shape=jax.ShapeDtypeStruct((N_PAD, 128), F32),
    )(x_p, W1, dinv)


def _tc_layer(o, g, dinv, b, W):
    return pl.pallas_call(
        _tc_layer_body,
        grid=(N_PAD // _BT,),
        in_specs=[
            pl.BlockSpec((NC, _BT, 128), lambda i: (0, i, 0)),
            pl.BlockSpec((_BT, 128), lambda i: (i, 0)),
            pl.BlockSpec((_BT, 1), lambda i: (i, 0)),
            pl.BlockSpec((1, 128), lambda i: (0, 0)),
            pl.BlockSpec((128, 128), lambda i: (0, 0)),
        ],
        out_specs=pl.BlockSpec((_BT, 128), lambda i: (i, 0)),
        out_shape=jax.ShapeDtypeStruct((N_PAD, 128), F32),
    )(o, g, dinv, b, W)


def _tc4(o, g, dinv, b, Wm):
    return pl.pallas_call(
        _tc4_body,
        grid=(N_PAD // _BT,),
        in_specs=[
            pl.BlockSpec((NC, _BT, 128), lambda i: (0, i, 0)),
            pl.BlockSpec((_BT, 128), lambda i: (i, 0)),
            pl.BlockSpec((_BT, 1), lambda i: (i, 0)),
            pl.BlockSpec((1, 128), lambda i: (0, 0)),
            pl.BlockSpec((128, 128), lambda i: (0, 0)),
        ],
        out_specs=pl.BlockSpec((_BT, 128), lambda i: (i, 0)),
        out_shape=jax.ShapeDtypeStruct((N_PAD, 128), F32),
    )(o, g, dinv, b, Wm)


def _tc5(S16, bm2):
    B2 = 4096
    return pl.pallas_call(
        _tc5_body,
        grid=(E2_PAD // B2,),
        in_specs=[
            pl.BlockSpec((B2, 16), lambda i: (i, 0)),
            pl.BlockSpec((1, 1), lambda i: (0, 0)),
        ],
        out_specs=pl.BlockSpec((B2, 1), lambda i: (i, 0)),
        out_shape=jax.ShapeDtypeStruct((E2_PAD, 1), F32),
    )(S16, bm2)


def kernel(x, edge_index, edge_label_index,
           W1, b1, W2, b2, W3, b3, Wm1, bm1, Wm2, bm2):
    ei = edge_index.astype(jnp.int32)
    eli = edge_label_index.astype(jnp.int32)

    pad_e = PAD_IDX + (jnp.arange(E_PAD - E, dtype=jnp.int32) % (N_PAD - N))
    src = jnp.concatenate([ei[0], pad_e]).reshape(NC * NS, NBLK, EBLK)
    dst = jnp.concatenate([ei[1], pad_e]).reshape(NC * NS, NBLK, EBLK)
    pad_p = PAD_IDX + (jnp.arange(E2_PAD - E2, dtype=jnp.int32) % (N_PAD - N))
    eli_p = jnp.concatenate(
        [eli, jnp.stack([pad_p, pad_p])],
        axis=1).reshape(2, NC, NS, NBLK2C, EBLK)
    x_p = jnp.pad(x, ((0, N_PAD - N), (0, 0)))
    W3p = jnp.pad(W3, ((0, 0), (0, 64)))
    b3p = jnp.pad(b3, (0, 64)).reshape(1, 128)
    Wmp = jnp.pad(jnp.concatenate([Wm1[:64], Wm1[64:]], axis=1),
                  ((0, 64), (0, 0)))

    degs = _deg(dst)
    dinv = (1.0 / jnp.sqrt(1.0 + degs[0] + degs[1])).reshape(N_PAD, 1)

    g1 = _tc1(x_p, W1, dinv)
    o1 = _agg(src, dst, g1)
    g2 = _tc_layer(o1, g1, dinv, b1.reshape(1, 128), W2)
    o2 = _agg(src, dst, g2)
    g3 = _tc_layer(o2, g2, dinv, b2.reshape(1, 128), W3p)
    o3 = _agg(src, dst, g3)
    T = _tc4(o3, g3, dinv, b3p, Wmp)
    S16 = _dec(eli_p[0], eli_p[1], T, bm1, Wm2[:, 0])
    logits = _tc5(S16, bm2.reshape(1, 1))
    return logits[:E2, 0]


# speedup vs baseline: 1.7147x; 1.1507x over previous
import functools

import jax
import jax.numpy as jnp
from jax import lax
from jax.experimental import pallas as pl
from jax.experimental.pallas import tpu as pltpu
from jax.experimental.pallas import tpu_sc as plsc

N = 10000
N_PAD = 10240
PAD_IDX = N
E = 320000
E_PAD = 327680
E2 = 100000
E2_PAD = 102400
EBLK = 128
NC, NS = 2, 16
RPT = N_PAD // NS
CHUNK = 16

_MESH = plsc.VectorSubcoreMesh(
    core_axis_name="c", subcore_axis_name="s", num_cores=NC, num_subcores=NS
)

F32 = jnp.float32


NBLK = E_PAD // (NC * NS) // EBLK
NCHUNK = NBLK // CHUNK
NBLK2 = E2_PAD // NS // EBLK


def _deg_body(dst_hbm, deg_out, deg_sh, ones_v, idx_v, zrow_v):
    c = lax.axis_index("c")
    s = lax.axis_index("s")
    for j in range(EBLK // 16):
        ones_v[pl.ds(j * 16, 16)] = jnp.ones((16,), F32)

    def zfill(i, carry):
        zrow_v[pl.ds(i * 16, 16)] = jnp.zeros((16,), F32)
        return carry

    lax.fori_loop(0, RPT // 16, zfill, 0)
    pltpu.sync_copy(zrow_v, deg_sh.at[pl.ds(s * RPT, RPT)])
    pltpu.sync_copy(dst_hbm.at[c * NS + s], idx_v)
    plsc.subcore_barrier()

    def body(i, carry):
        pltpu.sync_copy(ones_v, deg_sh.at[idx_v.at[i]], add=True)
        return carry

    lax.fori_loop(0, NBLK, body, 0)
    plsc.subcore_barrier()
    pltpu.sync_copy(deg_sh.at[pl.ds(s * RPT, RPT)],
                    deg_out.at[c, pl.ds(s * RPT, RPT)])


_deg = pl.kernel(
    _deg_body,
    out_type=jax.ShapeDtypeStruct((NC, N_PAD), F32),
    mesh=_MESH,
    scratch_types=[
        pltpu.VMEM_SHARED((N_PAD,), F32),
        pltpu.VMEM((EBLK,), F32),
        pltpu.VMEM((NBLK, EBLK), jnp.int32),
        pltpu.VMEM((RPT,), F32),
    ],
)


def _agg_body(src_hbm, dst_hbm, g_hbm, out_hbm, acc_sh,
              r0, r1, cs0, cs1, cd0, cd1,
              gsem0, gsem1, ssem0, ssem1, dsem0, dsem1):
    c = lax.axis_index("c")
    s = lax.axis_index("s")
    rows = (r0, r1)
    gsems = (gsem0, gsem1)
    csrc = (cs0, cs1)
    cdst = (cd0, cd1)
    ssems = (ssem0, ssem1)
    dsems = (dsem0, dsem1)
    wid = c * NS + s

    def zfill(i, carry):
        for j in range(128 // 16):
            r0[i, pl.ds(j * 16, 16)] = jnp.zeros((16,), F32)
        return carry

    lax.fori_loop(0, EBLK, zfill, 0)
    for k in range(RPT // EBLK):
        pltpu.sync_copy(r0, acc_sh.at[pl.ds(s * RPT + k * EBLK, EBLK)])
    pltpu.sync_copy(src_hbm.at[wid, pl.ds(0, CHUNK)], cs0)
    pltpu.sync_copy(dst_hbm.at[wid, pl.ds(0, CHUNK)], cd0)
    plsc.subcore_barrier()
    pltpu.async_copy(g_hbm.at[cs0.at[0]], r0, gsem0)

    def chunk_body(k, carry):
        for kp in range(2):
            @pl.when(k % 2 == kp)
            def _():
                @pl.when(k + 1 < NCHUNK)
                def _():
                    off = pl.multiple_of((k + 1) * CHUNK, CHUNK)
                    pltpu.async_copy(src_hbm.at[wid, pl.ds(off, CHUNK)],
                                     csrc[1 - kp], ssems[1 - kp])
                    pltpu.async_copy(dst_hbm.at[wid, pl.ds(off, CHUNK)],
                                     cdst[1 - kp], dsems[1 - kp])
                for j in range(CHUNK):
                    p = j % 2
                    if j + 1 < CHUNK:
                        pltpu.async_copy(g_hbm.at[csrc[kp].at[j + 1]],
                                         rows[1 - p], gsems[1 - p])
                    else:
                        @pl.when(k + 1 < NCHUNK)
                        def _():
                            pltpu.make_async_copy(
                                src_hbm.at[wid, pl.ds(0, CHUNK)],
                                csrc[1 - kp], ssems[1 - kp]).wait()
                            pltpu.make_async_copy(
                                dst_hbm.at[wid, pl.ds(0, CHUNK)],
                                cdst[1 - kp], dsems[1 - kp]).wait()
                            pltpu.async_copy(g_hbm.at[csrc[1 - kp].at[0]],
                                             rows[1 - p], gsems[1 - p])
                    pltpu.make_async_copy(g_hbm.at[csrc[kp].at[j]],
                                          rows[p], gsems[p]).wait()
                    pltpu.sync_copy(rows[p], acc_sh.at[cdst[kp].at[j]],
                                    add=True)
        return carry

    lax.fori_loop(0, NCHUNK, chunk_body, 0)
    plsc.subcore_barrier()
    pltpu.sync_copy(acc_sh.at[pl.ds(s * RPT, RPT)],
                    out_hbm.at[c, pl.ds(s * RPT, RPT)])


_agg = pl.kernel(
    _agg_body,
    out_type=jax.ShapeDtypeStruct((NC, N_PAD, 128), F32),
    mesh=_MESH,
    scratch_types=[
        pltpu.VMEM_SHARED((N_PAD, 128), F32),
        pltpu.VMEM((EBLK, 128), F32),
        pltpu.VMEM((EBLK, 128), F32),
        pltpu.VMEM((CHUNK, EBLK), jnp.int32),
        pltpu.VMEM((CHUNK, EBLK), jnp.int32),
        pltpu.VMEM((CHUNK, EBLK), jnp.int32),
        pltpu.VMEM((CHUNK, EBLK), jnp.int32),
        pltpu.SemaphoreType.DMA,
        pltpu.SemaphoreType.DMA,
        pltpu.SemaphoreType.DMA,
        pltpu.SemaphoreType.DMA,
        pltpu.SemaphoreType.DMA,
        pltpu.SemaphoreType.DMA,
    ],
)


NBLK2C = E2_PAD // (NC * NS) // EBLK

_GDN = lax.GatherDimensionNumbers(
    offset_dims=(), collapsed_slice_dims=(0,), start_index_map=(0,))


def _dec_body(e0_hbm, e1_hbm, t_hbm, bm1_hbm, wm2_hbm, bm2_hbm, out_hbm,
              ts0, ts1, td0, td1, o0, o1, i0_v, i1_v, bm1_v, wm2_v, bm2_v,
              sa0, sa1, sb0, sb1, w0, w1):
    c = lax.axis_index("c")
    s = lax.axis_index("s")
    ts = (ts0, ts1)
    td = (td0, td1)
    outs = (o0, o1)
    sas = (sa0, sa1)
    sbs = (sb0, sb1)
    ws = (w0, w1)
    base = (c * NS + s) * (NBLK2C * EBLK)

    pltpu.sync_copy(bm1_hbm, bm1_v)
    pltpu.sync_copy(wm2_hbm, wm2_v)
    pltpu.sync_copy(bm2_hbm, bm2_v)
    pltpu.sync_copy(e0_hbm.at[c, s], i0_v)
    pltpu.sync_copy(e1_hbm.at[c, s], i1_v)
    pltpu.async_copy(t_hbm.at[i0_v.at[0]], ts0, sa0)
    pltpu.async_copy(t_hbm.at[i1_v.at[0]], td0, sb0)

    def body(i, carry):
        for p in range(2):
            @pl.when(i % 2 == p)
            def _():
                @pl.when(i + 1 < NBLK2C)
                def _():
                    pltpu.async_copy(t_hbm.at[i0_v.at[i + 1]],
                                     ts[1 - p], sas[1 - p])
                    pltpu.async_copy(t_hbm.at[i1_v.at[i + 1]],
                                     td[1 - p], sbs[1 - p])
                pltpu.make_async_copy(t_hbm.at[i0_v.at[i]],
                                      ts[p], sas[p]).wait()
                pltpu.make_async_copy(t_hbm.at[i1_v.at[i]],
                                      td[p], sbs[p]).wait()

                @pl.when(i >= 2)
                def _():
                    pltpu.make_async_copy(
                        outs[p],
                        out_hbm.at[pl.ds(base, EBLK)], ws[p]).wait()

                bm2q = bm2_v[pl.ds(0, 16)]
                lanes = lax.iota(jnp.int32, 16)

                def group(g, carry2):
                    vec = jnp.zeros((16,), F32)
                    for u in range(16):
                        rr = g * 16 + u
                        acc = bm2q
                        for k in range(4):
                            a = ts[p][rr, pl.ds(k * 16, 16)]
                            b = td[p][rr, pl.ds(64 + k * 16, 16)]
                            h = jnp.maximum(
                                a + b + bm1_v[pl.ds(k * 16, 16)], 0.0)
                            acc = acc + h * wm2_v[pl.ds(k * 16, 16)]
                        for sh in (8, 4, 2, 1):
                            perm = jnp.reshape(lanes ^ sh, (16, 1))
                            acc = acc + lax.gather(
                                acc, perm, _GDN, slice_sizes=(1,),
                                mode=lax.GatherScatterMode.PROMISE_IN_BOUNDS)
                        vec = jnp.where(lanes == u, acc, vec)
                    outs[p][pl.ds(g * 16, 16)] = vec
                    return carry2

                lax.fori_loop(0, EBLK // 16, group, 0)
                pltpu.async_copy(
                    outs[p], out_hbm.at[pl.ds(base + i * EBLK, EBLK)], ws[p])
        return carry

    lax.fori_loop(0, NBLK2C, body, 0)
    pltpu.make_async_copy(o0, out_hbm.at[pl.ds(base, EBLK)], w0).wait()
    pltpu.make_async_copy(o1, out_hbm.at[pl.ds(base, EBLK)], w1).wait()


_dec = pl.kernel(
    _dec_body,
    out_type=jax.ShapeDtypeStruct((E2_PAD,), F32),
    mesh=_MESH,
    scratch_types=[
        pltpu.VMEM((EBLK, 128), F32),
        pltpu.VMEM((EBLK, 128), F32),
        pltpu.VMEM((EBLK, 128), F32),
        pltpu.VMEM((EBLK, 128), F32),
        pltpu.VMEM((EBLK,), F32),
        pltpu.VMEM((EBLK,), F32),
        pltpu.VMEM((NBLK2C, EBLK), jnp.int32),
        pltpu.VMEM((NBLK2C, EBLK), jnp.int32),
        pltpu.VMEM((64,), F32),
        pltpu.VMEM((64,), F32),
        pltpu.VMEM((16,), F32),
        pltpu.SemaphoreType.DMA,
        pltpu.SemaphoreType.DMA,
        pltpu.SemaphoreType.DMA,
        pltpu.SemaphoreType.DMA,
        pltpu.SemaphoreType.DMA,
        pltpu.SemaphoreType.DMA,
    ],
)


_BT = 1024


def _tc1_body(x_ref, w_ref, dinv_ref, o_ref):
    h = jnp.dot(x_ref[...], w_ref[...], preferred_element_type=F32)
    o_ref[...] = h * dinv_ref[...]


def _tc_layer_body(o_ref, g_ref, dinv_ref, b_ref, w_ref, out_ref):
    z = dinv_ref[...] * (o_ref[0] + o_ref[1] + g_ref[...]) + b_ref[...]
    z = jnp.maximum(z, 0.0)
    out_ref[...] = dinv_ref[...] * jnp.dot(
        z, w_ref[...], preferred_element_type=F32)


def _tc4_body(o_ref, g_ref, dinv_ref, b_ref, wm_ref, t_ref):
    z = dinv_ref[...] * (o_ref[0] + o_ref[1] + g_ref[...]) + b_ref[...]
    t_ref[...] = jnp.dot(z, wm_ref[...], preferred_element_type=F32)


def _tc5_body(s_ref, bm2_ref, o_ref):
    o_ref[...] = jnp.sum(s_ref[...], axis=1, keepdims=True) + bm2_ref[...]


def _tc1(x_p, W1, dinv):
    return pl.pallas_call(
        _tc1_body,
        grid=(N_PAD // _BT,),
        in_specs=[
            pl.BlockSpec((_BT, 128), lambda i: (i, 0)),
            pl.BlockSpec((128, 128), lambda i: (0, 0)),
            pl.BlockSpec((_BT, 1), lambda i: (i, 0)),
        ],
        out_specs=pl.BlockSpec((_BT, 128), lambda i: (i, 0)),
        out_shape=jax.ShapeDtypeStruct((N_PAD, 128), F32),
    )(x_p, W1, dinv)


def _tc_layer(o, g, dinv, b, W):
    return pl.pallas_call(
        _tc_layer_body,
        grid=(N_PAD // _BT,),
        in_specs=[
            pl.BlockSpec((NC, _BT, 128), lambda i: (0, i, 0)),
            pl.BlockSpec((_BT, 128), lambda i: (i, 0)),
            pl.BlockSpec((_BT, 1), lambda i: (i, 0)),
            pl.BlockSpec((1, 128), lambda i: (0, 0)),
            pl.BlockSpec((128, 128), lambda i: (0, 0)),
        ],
        out_specs=pl.BlockSpec((_BT, 128), lambda i: (i, 0)),
        out_shape=jax.ShapeDtypeStruct((N_PAD, 128), F32),
    )(o, g, dinv, b, W)


def _tc4(o, g, dinv, b, Wm):
    return pl.pallas_call(
        _tc4_body,
        grid=(N_PAD // _BT,),
        in_specs=[
            pl.BlockSpec((NC, _BT, 128), lambda i: (0, i, 0)),
            pl.BlockSpec((_BT, 128), lambda i: (i, 0)),
            pl.BlockSpec((_BT, 1), lambda i: (i, 0)),
            pl.BlockSpec((1, 128), lambda i: (0, 0)),
            pl.BlockSpec((128, 128), lambda i: (0, 0)),
        ],
        out_specs=pl.BlockSpec((_BT, 128), lambda i: (i, 0)),
        out_shape=jax.ShapeDtypeStruct((N_PAD, 128), F32),
    )(o, g, dinv, b, Wm)


def _tc5(S16, bm2):
    B2 = 4096
    return pl.pallas_call(
        _tc5_body,
        grid=(E2_PAD // B2,),
        in_specs=[
            pl.BlockSpec((B2, 16), lambda i: (i, 0)),
            pl.BlockSpec((1, 1), lambda i: (0, 0)),
        ],
        out_specs=pl.BlockSpec((B2, 1), lambda i: (i, 0)),
        out_shape=jax.ShapeDtypeStruct((E2_PAD, 1), F32),
    )(S16, bm2)


def kernel(x, edge_index, edge_label_index,
           W1, b1, W2, b2, W3, b3, Wm1, bm1, Wm2, bm2):
    ei = edge_index.astype(jnp.int32)
    eli = edge_label_index.astype(jnp.int32)

    pad_e = PAD_IDX + (jnp.arange(E_PAD - E, dtype=jnp.int32) % (N_PAD - N))
    src = jnp.concatenate([ei[0], pad_e]).reshape(NC * NS, NBLK, EBLK)
    dst = jnp.concatenate([ei[1], pad_e]).reshape(NC * NS, NBLK, EBLK)
    pad_p = PAD_IDX + (jnp.arange(E2_PAD - E2, dtype=jnp.int32) % (N_PAD - N))
    eli_p = jnp.concatenate(
        [eli, jnp.stack([pad_p, pad_p])],
        axis=1).reshape(2, NC, NS, NBLK2C, EBLK)
    x_p = jnp.pad(x, ((0, N_PAD - N), (0, 0)))
    W3p = jnp.pad(W3, ((0, 0), (0, 64)))
    b3p = jnp.pad(b3, (0, 64)).reshape(1, 128)
    Wmp = jnp.pad(jnp.concatenate([Wm1[:64], Wm1[64:]], axis=1),
                  ((0, 64), (0, 0)))

    degs = _deg(dst)
    dinv = (1.0 / jnp.sqrt(1.0 + degs[0] + degs[1])).reshape(N_PAD, 1)

    g1 = _tc1(x_p, W1, dinv)
    o1 = _agg(src, dst, g1)
    g2 = _tc_layer(o1, g1, dinv, b1.reshape(1, 128), W2)
    o2 = _agg(src, dst, g2)
    g3 = _tc_layer(o2, g2, dinv, b2.reshape(1, 128), W3p)
    o3 = _agg(src, dst, g3)
    T = _tc4(o3, g3, dinv, b3p, Wmp)
    bm2_16 = jnp.broadcast_to(bm2 / 16.0, (16,))
    logits = _dec(eli_p[0], eli_p[1], T, bm1, Wm2[:, 0], bm2_16)
    return logits[:E2]
